# baseline (speedup 1.0000x reference)
import functools

import jax
import jax.numpy as jnp
from jax import lax
from jax.experimental import pallas as pl
from jax.experimental.pallas import tpu as pltpu
from jax.experimental.pallas import tpu_sc as plsc

_B, _L, _K = 8, 2048, 48
_MAXREL = 32
_NCLS = 2 * _MAXREL + 1 + 1
_DOUT = 16
_LANES = 16
_NC, _NS = 2, 16
_NW = _NC * _NS
_WPB = _NW // _B
_LW = _L // _WPB
_LCH = _LW // _LANES
_KG = 4
_NG = _K // _KG
_DPAD = 80


def _sc_body(r_hbm, c_hbm, e_hbm, wt_hbm, b_hbm, out_hbm,
             r_v, c_v, m_v, t_v, tp_v, b_v, e_v, d_v, o_v, sem):
    cid = lax.axis_index("c")
    sid = lax.axis_index("s")
    wid = sid * _NC + cid
    bb = wid // _WPB
    l0 = (wid % _WPB) * _LW

    pltpu.sync_copy(r_hbm.at[bb], r_v)
    pltpu.sync_copy(c_hbm.at[bb], c_v)
    pltpu.sync_copy(e_hbm.at[bb, :, pl.ds(l0, _LW)], e_v)
    pltpu.sync_copy(wt_hbm, t_v)
    pltpu.sync_copy(b_hbm, b_v)

    bvec = b_v[...]
    for i in range(_NCLS):
        sl = pl.ds(i * _DOUT, _DOUT)
        t_v[sl] = t_v[sl] + bvec

    iota16 = lax.iota(jnp.int32, _LANES)
    for jj in range(_DOUT // 2):
        for cch in range(_DPAD // _LANES):
            dvec = jnp.minimum(cch * _LANES + iota16, _NCLS - 1) * _DOUT
            pa = plsc.load_gather(t_v, [dvec + (2 * jj)])
            pb = plsc.load_gather(t_v, [dvec + (2 * jj + 1)])
            w = plsc.pack(pa, pb, format=plsc.PackFormat.INTERLEAVED)
            tp_v[pl.ds(jj * _DPAD + cch * _LANES, _LANES)] = plsc.bitcast(
                w, jnp.int32)

    @plsc.parallel_loop(0, _L // _LANES, unroll=2)
    def _pack(i):
        sl = pl.ds(i * _LANES, _LANES)
        m_v[sl] = (r_v[sl] << 2) | c_v[sl]

    def compute_group(gk, slot):
        @plsc.parallel_loop(0, _KG * _LCH, unroll=2)
        def _cls(i):
            kg = lax.shift_right_logical(i, 5)
            lc = i & (_LCH - 1)
            l16 = lc * _LANES
            e = e_v[gk * _KG + kg, pl.ds(l16, _LANES)]
            ma = m_v[pl.ds(l0 + l16, _LANES)]
            mj = plsc.load_gather(m_v, [e])
            diff = lax.shift_right_arithmetic(ma - mj, 2)
            dd = jnp.clip(diff, -_MAXREL, _MAXREL) + _MAXREL
            dd = jnp.where(((ma ^ mj) & 3) == 0, dd, _NCLS - 1)
            d_v[pl.ds(i * _LANES, _LANES)] = dd

        @plsc.parallel_loop(0, _KG * _LCH, unroll=2)
        def _lc(i):
            kg = lax.shift_right_logical(i, 5)
            lc = i & (_LCH - 1)
            l16 = lc * _LANES
            dd = d_v[pl.ds(i * _LANES, _LANES)]
            for jj in range(_DOUT // 2):
                w = plsc.load_gather(tp_v, [dd + (jj * _DPAD)])
                pa, pb = plsc.unpack(plsc.bitcast(w, jnp.bfloat16),
                                     format=plsc.PackFormat.INTERLEAVED)
                o_v[slot, kg, 2 * jj, pl.ds(l16, _LANES)] = pa
                o_v[slot, kg, 2 * jj + 1, pl.ds(l16, _LANES)] = pb

    compute_group(0, 0)
    @pl.loop(1, _NG)
    def _g(gk):
        prev = (gk - 1) & 1
        cur = gk & 1
        cp = pltpu.async_copy(
            o_v.at[prev],
            out_hbm.at[bb, pl.ds((gk - 1) * _KG, _KG), :, pl.ds(l0, _LW)], sem)
        compute_group(gk, cur)
        cp.wait()
    pltpu.sync_copy(
        o_v.at[(_NG - 1) & 1],
        out_hbm.at[bb, pl.ds((_NG - 1) * _KG, _KG), :, pl.ds(l0, _LW)])


_sc_kernel = functools.partial(
    pl.kernel,
    out_type=jax.ShapeDtypeStruct((_B, _K, _DOUT, _L), jnp.float32),
    mesh=plsc.VectorSubcoreMesh(core_axis_name="c", subcore_axis_name="s",
                                num_cores=_NC, num_subcores=_NS),
    scratch_types=[
        pltpu.VMEM((_L,), jnp.int32),
        pltpu.VMEM((_L,), jnp.int32),
        pltpu.VMEM((_L,), jnp.int32),
        pltpu.VMEM((_NCLS * _DOUT,), jnp.float32),
        pltpu.VMEM((_DOUT // 2 * _DPAD,), jnp.int32),
        pltpu.VMEM((_DOUT,), jnp.float32),
        pltpu.VMEM((_K, _LW), jnp.int32),
        pltpu.VMEM((_KG * _LW,), jnp.int32),
        pltpu.VMEM((2, _KG, _DOUT, _LW), jnp.float32),
        pltpu.SemaphoreType.DMA,
    ],
    compiler_params=pltpu.CompilerParams(needs_layout_passes=False),
)(_sc_body)


def kernel(R_idx, chain_labels, E_idx, W, b):
    wt_flat = jnp.transpose(W).reshape(-1)
    e_t = jnp.transpose(E_idx, (0, 2, 1))
    out = _sc_kernel(R_idx, chain_labels, e_t, wt_flat, b)
    return jnp.transpose(out, (0, 3, 1, 2))


# speedup vs baseline: 1.0620x; 1.0620x over previous
import functools

import jax
import jax.numpy as jnp
from jax import lax
from jax.experimental import pallas as pl
from jax.experimental.pallas import tpu as pltpu
from jax.experimental.pallas import tpu_sc as plsc

_B, _L, _K = 8, 2048, 48
_MAXREL = 32
_NCLS = 2 * _MAXREL + 1 + 1
_DOUT = 16
_LANES = 16
_NC, _NS = 2, 16
_NW = _NC * _NS
_WPB = _NW // _B
_LW = _L // _WPB
_LCH = _LW // _LANES
_KG = 4
_NG = _K // _KG
_DPAD = 80


def _sc_body(r_hbm, c_hbm, e_hbm, wt_hbm, b_hbm, out_hbm,
             r_v, c_v, m_v, t_v, tp_v, b_v, e_v, o_v, sem):
    cid = lax.axis_index("c")
    sid = lax.axis_index("s")
    wid = sid * _NC + cid
    bb = wid // _WPB
    l0 = (wid % _WPB) * _LW

    e_cp = pltpu.async_copy(e_hbm.at[bb, :, pl.ds(l0, _LW)], e_v, sem)
    pltpu.sync_copy(r_hbm.at[bb], r_v)
    pltpu.sync_copy(c_hbm.at[bb], c_v)
    pltpu.sync_copy(wt_hbm, t_v)
    pltpu.sync_copy(b_hbm, b_v)

    bvec = b_v[...]
    for i in range(_NCLS):
        sl = pl.ds(i * _DOUT, _DOUT)
        t_v[sl] = t_v[sl] + bvec

    iota16 = lax.iota(jnp.int32, _LANES)
    for jj in range(_DOUT // 2):
        for cch in range(_DPAD // _LANES):
            dvec = jnp.minimum(cch * _LANES + iota16, _NCLS - 1) * _DOUT
            pa = plsc.load_gather(t_v, [dvec + (2 * jj)])
            pb = plsc.load_gather(t_v, [dvec + (2 * jj + 1)])
            w = plsc.pack(pa, pb, format=plsc.PackFormat.INTERLEAVED)
            tp_v[pl.ds(jj * _DPAD + cch * _LANES, _LANES)] = plsc.bitcast(
                w, jnp.int32)

    @plsc.parallel_loop(0, _L // _LANES, unroll=2)
    def _pack(i):
        sl = pl.ds(i * _LANES, _LANES)
        m_v[sl] = (r_v[sl] << 2) | c_v[sl]

    def compute_group(gk, slot):
        @plsc.parallel_loop(0, _KG * _LCH, unroll=2)
        def _lc(i):
            kg = lax.shift_right_logical(i, 5)
            lc = i & (_LCH - 1)
            l16 = lc * _LANES
            e = e_v[gk * _KG + kg, pl.ds(l16, _LANES)]
            ma = m_v[pl.ds(l0 + l16, _LANES)]
            mj = plsc.load_gather(m_v, [e])
            diff = lax.shift_right_arithmetic(ma - mj, 2)
            dd = jnp.clip(diff, -_MAXREL, _MAXREL) + _MAXREL
            dd = jnp.where(((ma ^ mj) & 3) == 0, dd, _NCLS - 1)
            for jj in range(_DOUT // 2):
                w = plsc.load_gather(tp_v, [dd + (jj * _DPAD)])
                pa, pb = plsc.unpack(plsc.bitcast(w, jnp.bfloat16),
                                     format=plsc.PackFormat.INTERLEAVED)
                o_v[slot, kg, 2 * jj, pl.ds(l16, _LANES)] = pa
                o_v[slot, kg, 2 * jj + 1, pl.ds(l16, _LANES)] = pb

    e_cp.wait()
    compute_group(0, 0)
    @pl.loop(1, _NG)
    def _g(gk):
        prev = (gk - 1) & 1
        cur = gk & 1
        cp = pltpu.async_copy(
            o_v.at[prev],
            out_hbm.at[bb, pl.ds((gk - 1) * _KG, _KG), :, pl.ds(l0, _LW)], sem)
        compute_group(gk, cur)
        cp.wait()
    pltpu.sync_copy(
        o_v.at[(_NG - 1) & 1],
        out_hbm.at[bb, pl.ds((_NG - 1) * _KG, _KG), :, pl.ds(l0, _LW)])


_sc_kernel = functools.partial(
    pl.kernel,
    out_type=jax.ShapeDtypeStruct((_B, _K, _DOUT, _L), jnp.float32),
    mesh=plsc.VectorSubcoreMesh(core_axis_name="c", subcore_axis_name="s",
                                num_cores=_NC, num_subcores=_NS),
    scratch_types=[
        pltpu.VMEM((_L,), jnp.int32),
        pltpu.VMEM((_L,), jnp.int32),
        pltpu.VMEM((_L,), jnp.int32),
        pltpu.VMEM((_NCLS * _DOUT,), jnp.float32),
        pltpu.VMEM((_DOUT // 2 * _DPAD,), jnp.int32),
        pltpu.VMEM((_DOUT,), jnp.float32),
        pltpu.VMEM((_K, _LW), jnp.int32),
        pltpu.VMEM((2, _KG, _DOUT, _LW), jnp.float32),
        pltpu.SemaphoreType.DMA,
    ],
    compiler_params=pltpu.CompilerParams(needs_layout_passes=False),
)(_sc_body)


def kernel(R_idx, chain_labels, E_idx, W, b):
    wt_flat = jnp.transpose(W).reshape(-1)
    e_t = jnp.transpose(E_idx, (0, 2, 1))
    out = _sc_kernel(R_idx, chain_labels, e_t, wt_flat, b)
    return jnp.transpose(out, (0, 3, 1, 2))


# speedup vs baseline: 1.0642x; 1.0020x over previous
import functools

import jax
import jax.numpy as jnp
from jax import lax
from jax.experimental import pallas as pl
from jax.experimental.pallas import tpu as pltpu
from jax.experimental.pallas import tpu_sc as plsc

_B, _L, _K = 8, 2048, 48
_MAXREL = 32
_NCLS = 2 * _MAXREL + 1 + 1
_DOUT = 16
_LANES = 16
_NC, _NS = 2, 16
_NW = _NC * _NS
_WPB = _NW // _B
_LW = _L // _WPB
_LCH = _LW // _LANES
_KG = 4
_NG = _K // _KG
_DPAD = 80


def _sc_body(r_hbm, c_hbm, e_hbm, wt_hbm, b_hbm, out_hbm,
             r_v, c_v, m_v, t_v, tp_v, b_v, e_v, o_v, sem):
    cid = lax.axis_index("c")
    sid = lax.axis_index("s")
    wid = sid * _NC + cid
    bb = wid // _WPB
    l0 = (wid % _WPB) * _LW

    e_cp = pltpu.async_copy(e_hbm.at[bb, :, pl.ds(l0, _LW)], e_v, sem)
    pltpu.sync_copy(r_hbm.at[bb], r_v)
    pltpu.sync_copy(c_hbm.at[bb], c_v)
    pltpu.sync_copy(wt_hbm, t_v)
    pltpu.sync_copy(b_hbm, b_v)

    bvec = b_v[...]
    for i in range(_NCLS):
        sl = pl.ds(i * _DOUT, _DOUT)
        t_v[sl] = t_v[sl] + bvec

    iota16 = lax.iota(jnp.int32, _LANES)
    for jj in range(_DOUT // 2):
        for cch in range(_DPAD // _LANES):
            dvec = jnp.minimum(cch * _LANES + iota16, _NCLS - 1) * _DOUT
            pa = plsc.load_gather(t_v, [dvec + (2 * jj)])
            pb = plsc.load_gather(t_v, [dvec + (2 * jj + 1)])
            w = plsc.pack(pa, pb, format=plsc.PackFormat.INTERLEAVED)
            tp_v[pl.ds(jj * _DPAD + cch * _LANES, _LANES)] = plsc.bitcast(
                w, jnp.int32)

    @plsc.parallel_loop(0, _L // _LANES, unroll=2)
    def _pack(i):
        sl = pl.ds(i * _LANES, _LANES)
        m_v[sl] = (r_v[sl] << 2) | c_v[sl]

    def compute_group(gk, slot):
        @plsc.parallel_loop(0, _KG * _LCH, unroll=2)
        def _lc(i):
            kg = lax.shift_right_logical(i, 5)
            lc = i & (_LCH - 1)
            l16 = lc * _LANES
            e = e_v[gk * _KG + kg, pl.ds(l16, _LANES)]
            ma = m_v[pl.ds(l0 + l16, _LANES)]
            mj = plsc.load_gather(m_v, [e])
            diff = lax.shift_right_arithmetic(ma - mj, 2)
            dd = jnp.clip(diff, -_MAXREL, _MAXREL) + _MAXREL
            dd = jnp.where(((ma ^ mj) & 3) == 0, dd, _NCLS - 1)
            for jj in range(_DOUT // 2):
                w = plsc.load_gather(tp_v, [dd + (jj * _DPAD)])
                pa, pb = plsc.unpack(plsc.bitcast(w, jnp.bfloat16),
                                     format=plsc.PackFormat.INTERLEAVED)
                o_v[slot, kg, 2 * jj, pl.ds(l16, _LANES)] = pa
                o_v[slot, kg, 2 * jj + 1, pl.ds(l16, _LANES)] = pb

    e_cp.wait()
    compute_group(0, 0)
    @pl.loop(1, _NG)
    def _g(gk):
        prev = (gk - 1) & 1
        cur = gk & 1
        cp = pltpu.async_copy(
            o_v.at[prev],
            out_hbm.at[bb, pl.ds((gk - 1) * _KG, _KG), :, pl.ds(l0, _LW)], sem)
        compute_group(gk, cur)
        cp.wait()
    pltpu.sync_copy(
        o_v.at[(_NG - 1) & 1],
        out_hbm.at[bb, pl.ds((_NG - 1) * _KG, _KG), :, pl.ds(l0, _LW)])


_sc_kernel = functools.partial(
    pl.kernel,
    out_type=jax.ShapeDtypeStruct((_B, _K, _DOUT, _L), jnp.float32),
    mesh=plsc.VectorSubcoreMesh(core_axis_name="c", subcore_axis_name="s",
                                num_cores=_NC, num_subcores=_NS),
    scratch_types=[
        pltpu.VMEM((_L,), jnp.int32),
        pltpu.VMEM((_L,), jnp.int32),
        pltpu.VMEM((_L,), jnp.int32),
        pltpu.VMEM((_NCLS * _DOUT,), jnp.float32),
        pltpu.VMEM((_DOUT // 2 * _DPAD,), jnp.int32),
        pltpu.VMEM((_DOUT,), jnp.float32),
        pltpu.VMEM((_K, _LW), jnp.int32),
        pltpu.VMEM((2, _KG, _DOUT, _LW), jnp.float32),
        pltpu.SemaphoreType.DMA,
    ],
    compiler_params=pltpu.CompilerParams(needs_layout_passes=False, skip_device_barrier=True),
)(_sc_body)


def kernel(R_idx, chain_labels, E_idx, W, b):
    wt_flat = jnp.transpose(W).reshape(-1)
    e_t = jnp.transpose(E_idx, (0, 2, 1))
    out = _sc_kernel(R_idx, chain_labels, e_t, wt_flat, b)
    return jnp.transpose(out, (0, 3, 1, 2))


# speedup vs baseline: 1.0661x; 1.0018x over previous
import functools

import jax
import jax.numpy as jnp
from jax import lax
from jax.experimental import pallas as pl
from jax.experimental.pallas import tpu as pltpu
from jax.experimental.pallas import tpu_sc as plsc

_B, _L, _K = 8, 2048, 48
_MAXREL = 32
_NCLS = 2 * _MAXREL + 1 + 1
_DOUT = 16
_LANES = 16
_NC, _NS = 2, 16
_NW = _NC * _NS
_WPB = _NW // _B
_LW = _L // _WPB
_LCH = _LW // _LANES
_KG = 4
_NG = _K // _KG
_DPAD = 80


def _sc_body(r_hbm, c_hbm, e_hbm, w_hbm, b_hbm, out_hbm,
             r_v, c_v, m_v, w_v, tp_v, b_v, e_v, o_v, sem):
    cid = lax.axis_index("c")
    sid = lax.axis_index("s")
    wid = sid * _NC + cid
    bb = wid // _WPB
    l0 = (wid % _WPB) * _LW

    e_cp = pltpu.async_copy(e_hbm.at[bb, :, pl.ds(l0, _LW)], e_v, sem)
    pltpu.sync_copy(r_hbm.at[bb], r_v)
    pltpu.sync_copy(c_hbm.at[bb], c_v)
    pltpu.sync_copy(w_hbm, w_v)
    pltpu.sync_copy(b_hbm, b_v)

    iota16 = lax.iota(jnp.int32, _LANES)
    for jj in range(_DOUT // 2):
        j0 = jnp.full((_LANES,), 2 * jj, dtype=jnp.int32)
        j1 = jnp.full((_LANES,), 2 * jj + 1, dtype=jnp.int32)
        b0 = plsc.load_gather(b_v, [j0])
        b1 = plsc.load_gather(b_v, [j1])
        for cch in range(_DPAD // _LANES):
            dvec = jnp.minimum(cch * _LANES + iota16, _NCLS - 1)
            pa = plsc.load_gather(w_v, [j0, dvec]) + b0
            pb = plsc.load_gather(w_v, [j1, dvec]) + b1
            w = plsc.pack(pa, pb, format=plsc.PackFormat.INTERLEAVED)
            tp_v[pl.ds(jj * _DPAD + cch * _LANES, _LANES)] = plsc.bitcast(
                w, jnp.int32)

    @plsc.parallel_loop(0, _L // _LANES, unroll=2)
    def _pack(i):
        sl = pl.ds(i * _LANES, _LANES)
        m_v[sl] = (r_v[sl] << 2) | c_v[sl]

    def compute_group(gk, slot):
        @plsc.parallel_loop(0, _KG * _LCH, unroll=2)
        def _lc(i):
            kg = lax.shift_right_logical(i, 5)
            lc = i & (_LCH - 1)
            l16 = lc * _LANES
            e = e_v[gk * _KG + kg, pl.ds(l16, _LANES)]
            ma = m_v[pl.ds(l0 + l16, _LANES)]
            mj = plsc.load_gather(m_v, [e])
            diff = lax.shift_right_arithmetic(ma - mj, 2)
            dd = jnp.clip(diff, -_MAXREL, _MAXREL) + _MAXREL
            dd = jnp.where(((ma ^ mj) & 3) == 0, dd, _NCLS - 1)
            for jj in range(_DOUT // 2):
                w = plsc.load_gather(tp_v, [dd + (jj * _DPAD)])
                pa, pb = plsc.unpack(plsc.bitcast(w, jnp.bfloat16),
                                     format=plsc.PackFormat.INTERLEAVED)
                o_v[slot, kg, 2 * jj, pl.ds(l16, _LANES)] = pa
                o_v[slot, kg, 2 * jj + 1, pl.ds(l16, _LANES)] = pb

    e_cp.wait()
    compute_group(0, 0)
    @pl.loop(1, _NG)
    def _g(gk):
        prev = (gk - 1) & 1
        cur = gk & 1
        cp = pltpu.async_copy(
            o_v.at[prev],
            out_hbm.at[bb, pl.ds((gk - 1) * _KG, _KG), :, pl.ds(l0, _LW)], sem)
        compute_group(gk, cur)
        cp.wait()
    pltpu.sync_copy(
        o_v.at[(_NG - 1) & 1],
        out_hbm.at[bb, pl.ds((_NG - 1) * _KG, _KG), :, pl.ds(l0, _LW)])


_sc_kernel = functools.partial(
    pl.kernel,
    out_type=jax.ShapeDtypeStruct((_B, _K, _DOUT, _L), jnp.float32),
    mesh=plsc.VectorSubcoreMesh(core_axis_name="c", subcore_axis_name="s",
                                num_cores=_NC, num_subcores=_NS),
    scratch_types=[
        pltpu.VMEM((_L,), jnp.int32),
        pltpu.VMEM((_L,), jnp.int32),
        pltpu.VMEM((_L,), jnp.int32),
        pltpu.VMEM((_DOUT, _NCLS), jnp.float32),
        pltpu.VMEM((_DOUT // 2 * _DPAD,), jnp.int32),
        pltpu.VMEM((_DOUT,), jnp.float32),
        pltpu.VMEM((_K, _LW), jnp.int32),
        pltpu.VMEM((2, _KG, _DOUT, _LW), jnp.float32),
        pltpu.SemaphoreType.DMA,
    ],
    compiler_params=pltpu.CompilerParams(needs_layout_passes=False),
)(_sc_body)


def kernel(R_idx, chain_labels, E_idx, W, b):
    e_t = jnp.transpose(E_idx, (0, 2, 1))
    out = _sc_kernel(R_idx, chain_labels, e_t, W, b)
    return jnp.transpose(out, (0, 3, 1, 2))
